# baseline (speedup 1.0000x reference)
import functools

import jax
import jax.numpy as jnp
from jax import lax
from jax.experimental import pallas as pl
from jax.experimental.pallas import tpu as pltpu
from jax.experimental.pallas import tpu_sc as plsc

B = 4096
MF_DIM = 32
WORD_DIM = 64
DOC_LEN = 200
HALF = 100
NC = 2
NS = 16
NW = NC * NS
BPW = B // NW
UNROLL = 8
SPLIT_A = 96
SPLIT_B = 104


def _doc_body(udoc_h, idoc_h, doc_h, ud_o, id_o,
              didx_v, rows_v, dout_v, sem0, sem1):
    wid = lax.axis_index("s") * NC + lax.axis_index("c")
    base = wid * BPW
    inv_len = jnp.float32(1.0 / DOC_LEN)
    sems = (sem0, sem1)

    def do_doc(doc_idx_h, out_h):
        pltpu.sync_copy(doc_idx_h.at[pl.ds(base, BPW)], didx_v)

        def fire(i, b):
            pltpu.async_copy(doc_h.at[didx_v.at[i, pl.ds(0, SPLIT_A)]],
                             rows_v.at[b, pl.ds(0, SPLIT_A)], sems[b])
            pltpu.async_copy(doc_h.at[didx_v.at[i, pl.ds(SPLIT_A, SPLIT_B)]],
                             rows_v.at[b, pl.ds(SPLIT_A, SPLIT_B)], sems[b])

        def drain(b):
            pltpu.make_async_copy(doc_h.at[didx_v.at[0, pl.ds(0, SPLIT_A)]],
                                  rows_v.at[b], sems[b]).wait()

        def reduce_store(i, b):
            def red(r2, acc):
                a0, a1, a2, a3 = acc
                r0 = r2 * UNROLL
                for rr in range(UNROLL):
                    a0 = a0 + rows_v[b, r0 + rr, pl.ds(0, 16)]
                    a1 = a1 + rows_v[b, r0 + rr, pl.ds(16, 16)]
                    a2 = a2 + rows_v[b, r0 + rr, pl.ds(32, 16)]
                    a3 = a3 + rows_v[b, r0 + rr, pl.ds(48, 16)]
                return (a0, a1, a2, a3)

            z = jnp.zeros((16,), jnp.float32)
            a0, a1, a2, a3 = lax.fori_loop(0, DOC_LEN // UNROLL, red,
                                           (z, z, z, z))
            dout_v[i, pl.ds(0, 16)] = a0 * inv_len
            dout_v[i, pl.ds(16, 16)] = a1 * inv_len
            dout_v[i, pl.ds(32, 16)] = a2 * inv_len
            dout_v[i, pl.ds(48, 16)] = a3 * inv_len

        fire(0, 0)
        fire(1, 1)

        def pair(g, carry):
            for b in range(2):
                i = 2 * g + b
                drain(b)
                reduce_store(i, b)
                fire(i + 2, b)
            return carry

        lax.fori_loop(0, BPW // 2 - 1, pair, jnp.int32(0))
        for b in range(2):
            drain(b)
            reduce_store(BPW - 2 + b, b)

        pltpu.sync_copy(dout_v, out_h.at[pl.ds(base, BPW)])

    do_doc(udoc_h, ud_o)
    do_doc(idoc_h, id_o)


CHUNK = 1024
BPS = B // NC


GRP = 128 // MF_DIM


def _mf_body(user_h, item_h, gu_h, gi_h, tu_h, ti_h,
             gu_o, gi_o, tu_o, ti_o,
             idx_s, stage_sh, sem, osem):
    cid = lax.axis_index("c")
    base = cid * BPS

    tables = (gu_h, tu_h, gi_h, ti_h)
    outs = (gu_o, tu_o, gi_o, ti_o)
    idx_arrs = (user_h, user_h, item_h, item_h)

    for t in range(4):
        for c in range(BPS // CHUNK):
            off = base + c * CHUNK
            pltpu.sync_copy(idx_arrs[t].at[pl.ds(off, CHUNK)], idx_s)

            def fetch(i, carry):
                row = idx_s[i] // GRP
                pltpu.async_copy(tables[t].at[pl.ds(row, 1)],
                                 stage_sh.at[pl.ds(i, 1)], sem)
                return carry

            lax.fori_loop(0, CHUNK, fetch, jnp.int32(0))

            def drain(i, carry):
                pltpu.make_async_copy(tables[t].at[pl.ds(0, 1)],
                                      stage_sh.at[pl.ds(0, 1)], sem).wait()
                return carry

            lax.fori_loop(0, CHUNK, drain, jnp.int32(0))
            pltpu.async_copy(stage_sh, outs[t].at[pl.ds(off, CHUNK)], osem)
            pltpu.make_async_copy(stage_sh, outs[t].at[pl.ds(off, CHUNK)],
                                  osem).wait()


@jax.jit
def kernel(user, item, user_doc, item_doc, gamma_user_w, gamma_item_w,
           theta_user_w, theta_item_w, doc_w):
    mesh = plsc.VectorSubcoreMesh(core_axis_name="c", subcore_axis_name="s")

    doc_f = pl.kernel(
        _doc_body,
        out_type=(
            jax.ShapeDtypeStruct((B, WORD_DIM), jnp.float32),
            jax.ShapeDtypeStruct((B, WORD_DIM), jnp.float32),
        ),
        mesh=mesh,
        compiler_params=pltpu.CompilerParams(use_tc_tiling_on_sc=False),
        scratch_types=[
            pltpu.VMEM((BPW, DOC_LEN), jnp.int32),
            pltpu.VMEM((2, DOC_LEN, WORD_DIM), jnp.float32),
            pltpu.VMEM((BPW, WORD_DIM), jnp.float32),
            pltpu.SemaphoreType.DMA,
            pltpu.SemaphoreType.DMA,
        ],
    )
    ud_e, id_e = doc_f(user_doc, item_doc, doc_w)

    mf_f = pl.kernel(
        _mf_body,
        out_type=(
            jax.ShapeDtypeStruct((B, 128), jnp.float32),
            jax.ShapeDtypeStruct((B, 128), jnp.float32),
            jax.ShapeDtypeStruct((B, 128), jnp.float32),
            jax.ShapeDtypeStruct((B, 128), jnp.float32),
        ),
        mesh=plsc.ScalarSubcoreMesh(axis_name="c", num_cores=NC),
        compiler_params=pltpu.CompilerParams(use_tc_tiling_on_sc=True),
        scratch_types=[
            pltpu.SMEM((CHUNK,), jnp.int32),
            pltpu.VMEM_SHARED((CHUNK, 128), jnp.float32),
            pltpu.SemaphoreType.DMA,
            pltpu.SemaphoreType.DMA,
        ],
    )
    tok = (ud_e[0, 0] * 0.0).astype(jnp.int32) + (id_e[0, 0] * 0.0).astype(jnp.int32)
    gu_p, gi_p, tu_p, ti_p = mf_f(
        user + tok, item + tok,
        gamma_user_w.reshape(-1, 128), gamma_item_w.reshape(-1, 128),
        theta_user_w.reshape(-1, 128), theta_item_w.reshape(-1, 128))

    def pick(packed, idx):
        grp = (idx % GRP).astype(jnp.int32)
        onehot = (grp[:, None] == jnp.arange(GRP, dtype=jnp.int32)[None, :])
        r = packed.reshape(B, GRP, MF_DIM)
        return jnp.sum(r * onehot[:, :, None].astype(jnp.float32), axis=1)

    return (pick(gu_p, user), pick(gi_p, item),
            pick(tu_p, user), pick(ti_p, item), ud_e, id_e)


# speedup vs baseline: 1.3069x; 1.3069x over previous
import functools

import jax
import jax.numpy as jnp
from jax import lax
from jax.experimental import pallas as pl
from jax.experimental.pallas import tpu as pltpu
from jax.experimental.pallas import tpu_sc as plsc

B = 4096
MF_DIM = 32
WORD_DIM = 64
DOC_LEN = 200
HALF = 100
NC = 2
NS = 16
NW = NC * NS
BPW = B // NW
UNROLL = 8
SPLIT_A = 96
SPLIT_B = 104


def _doc_body(udoc_h, idoc_h, doc_h, ud_o, id_o,
              didx_v, rows_v, dout_v, sem0, sem1):
    wid = lax.axis_index("s") * NC + lax.axis_index("c")
    base = wid * BPW
    inv_len = jnp.float32(1.0 / DOC_LEN)
    sems = (sem0, sem1)

    def do_doc(doc_idx_h, out_h):
        pltpu.sync_copy(doc_idx_h.at[pl.ds(base, BPW)], didx_v)

        def fire(i, b):
            pltpu.async_copy(doc_h.at[didx_v.at[i, pl.ds(0, SPLIT_A)]],
                             rows_v.at[b, pl.ds(0, SPLIT_A)], sems[b])
            pltpu.async_copy(doc_h.at[didx_v.at[i, pl.ds(SPLIT_A, SPLIT_B)]],
                             rows_v.at[b, pl.ds(SPLIT_A, SPLIT_B)], sems[b])

        def drain(b):
            pltpu.make_async_copy(doc_h.at[didx_v.at[0, pl.ds(0, SPLIT_A)]],
                                  rows_v.at[b], sems[b]).wait()

        def reduce_store(i, b):
            def red(r2, acc):
                a0, a1, a2, a3 = acc
                r0 = r2 * UNROLL
                for rr in range(UNROLL):
                    a0 = a0 + rows_v[b, r0 + rr, pl.ds(0, 16)]
                    a1 = a1 + rows_v[b, r0 + rr, pl.ds(16, 16)]
                    a2 = a2 + rows_v[b, r0 + rr, pl.ds(32, 16)]
                    a3 = a3 + rows_v[b, r0 + rr, pl.ds(48, 16)]
                return (a0, a1, a2, a3)

            z = jnp.zeros((16,), jnp.float32)
            a0, a1, a2, a3 = lax.fori_loop(0, DOC_LEN // UNROLL, red,
                                           (z, z, z, z))
            dout_v[i, pl.ds(0, 16)] = a0 * inv_len
            dout_v[i, pl.ds(16, 16)] = a1 * inv_len
            dout_v[i, pl.ds(32, 16)] = a2 * inv_len
            dout_v[i, pl.ds(48, 16)] = a3 * inv_len

        fire(0, 0)
        fire(1, 1)

        def pair(g, carry):
            for b in range(2):
                i = 2 * g + b
                drain(b)
                reduce_store(i, b)
                fire(i + 2, b)
            return carry

        lax.fori_loop(0, BPW // 2 - 1, pair, jnp.int32(0))
        for b in range(2):
            drain(b)
            reduce_store(BPW - 2 + b, b)

        pltpu.sync_copy(dout_v, out_h.at[pl.ds(base, BPW)])

    do_doc(udoc_h, ud_o)
    do_doc(idoc_h, id_o)


CHUNK = 1024
BPS = B // NC


def _mf_body(user_h, item_h, gu_h, gi_h, tu_h, ti_h,
             gu_o, gi_o, tu_o, ti_o,
             idx_s, stage_sh, sem, osem):
    cid = lax.axis_index("c")
    base = cid * BPS

    tables = (gu_h, tu_h, gi_h, ti_h)
    outs = (gu_o, tu_o, gi_o, ti_o)
    idx_arrs = (user_h, user_h, item_h, item_h)

    for t in range(4):
        for c in range(BPS // CHUNK):
            off = base + c * CHUNK
            pltpu.sync_copy(idx_arrs[t].at[pl.ds(off, CHUNK)], idx_s)

            def fetch(i, carry):
                idx = idx_s[i]
                pltpu.async_copy(tables[t].at[pl.ds(idx, 1)],
                                 stage_sh.at[pl.ds(i, 1)], sem)
                return carry

            lax.fori_loop(0, CHUNK, fetch, jnp.int32(0))

            def drain(i, carry):
                pltpu.make_async_copy(tables[t].at[pl.ds(0, 1)],
                                      stage_sh.at[pl.ds(0, 1)], sem).wait()
                return carry

            lax.fori_loop(0, CHUNK, drain, jnp.int32(0))
            pltpu.async_copy(stage_sh, outs[t].at[pl.ds(off, CHUNK)], osem)
            pltpu.make_async_copy(stage_sh, outs[t].at[pl.ds(off, CHUNK)],
                                  osem).wait()


@jax.jit
def kernel(user, item, user_doc, item_doc, gamma_user_w, gamma_item_w,
           theta_user_w, theta_item_w, doc_w):
    mesh = plsc.VectorSubcoreMesh(core_axis_name="c", subcore_axis_name="s")

    doc_f = pl.kernel(
        _doc_body,
        out_type=(
            jax.ShapeDtypeStruct((B, WORD_DIM), jnp.float32),
            jax.ShapeDtypeStruct((B, WORD_DIM), jnp.float32),
        ),
        mesh=mesh,
        compiler_params=pltpu.CompilerParams(use_tc_tiling_on_sc=False),
        scratch_types=[
            pltpu.VMEM((BPW, DOC_LEN), jnp.int32),
            pltpu.VMEM((2, DOC_LEN, WORD_DIM), jnp.float32),
            pltpu.VMEM((BPW, WORD_DIM), jnp.float32),
            pltpu.SemaphoreType.DMA,
            pltpu.SemaphoreType.DMA,
        ],
    )
    ud_e, id_e = doc_f(user_doc, item_doc, doc_w)

    mf_f = pl.kernel(
        _mf_body,
        out_type=(
            jax.ShapeDtypeStruct((B, MF_DIM), jnp.float32),
            jax.ShapeDtypeStruct((B, MF_DIM), jnp.float32),
            jax.ShapeDtypeStruct((B, MF_DIM), jnp.float32),
            jax.ShapeDtypeStruct((B, MF_DIM), jnp.float32),
        ),
        mesh=plsc.ScalarSubcoreMesh(axis_name="c", num_cores=NC),
        compiler_params=pltpu.CompilerParams(use_tc_tiling_on_sc=True),
        scratch_types=[
            pltpu.SMEM((CHUNK,), jnp.int32),
            pltpu.VMEM_SHARED((CHUNK, MF_DIM), jnp.float32),
            pltpu.SemaphoreType.DMA,
            pltpu.SemaphoreType.DMA,
        ],
    )
    tok = (ud_e[0, 0] * 0.0).astype(jnp.int32) + (id_e[0, 0] * 0.0).astype(jnp.int32)
    gu_e, gi_e, tu_e, ti_e = mf_f(user + tok, item + tok,
                                  gamma_user_w, gamma_item_w,
                                  theta_user_w, theta_item_w)
    return (gu_e, gi_e, tu_e, ti_e, ud_e, id_e)
